# speedup vs baseline: 1.1319x; 1.1319x over previous
import functools

import jax
import jax.numpy as jnp
from jax import lax
from jax.experimental import pallas as pl
from jax.experimental.pallas import tpu as pltpu
from jax.experimental.pallas import tpu_sc as plsc

B = 16384
D = 128
S = 5

_info = plsc.get_sparse_core_info()
_NC = _info.num_cores
_NS = _info.num_subcores
_NW = _NC * _NS

_C = 128


def _build():
    bpw = B // _NW
    npw = (B * S) // _NW

    mesh = plsc.VectorSubcoreMesh(core_axis_name="c", subcore_axis_name="s")

    @functools.partial(
        pl.kernel,
        mesh=mesh,
        out_type=(
            jax.ShapeDtypeStruct((B, D), jnp.float32),
            jax.ShapeDtypeStruct((B, D), jnp.float32),
            jax.ShapeDtypeStruct((B * S, D), jnp.float32),
        ),
        scratch_types=[
            pltpu.VMEM((_C,), jnp.int32),
            pltpu.VMEM((_C, D), jnp.float32),
            pltpu.SemaphoreType.DMA,
        ],
    )
    def k(iw_hbm, ow_hbm, nw_hbm, inp_hbm, oemb_hbm,
          o_inp, o_out, o_noise, idx_v, rows_v, sem):
        wid = lax.axis_index("s") * _NC + lax.axis_index("c")

        def gather(idx_hbm, table_hbm, out_hbm, per_w):
            base = wid * per_w

            def body(i, carry):
                off = base + i * _C
                pltpu.sync_copy(idx_hbm.at[pl.ds(off, _C)], idx_v)
                pltpu.async_copy(table_hbm.at[idx_v], rows_v, sem).wait()
                pltpu.sync_copy(rows_v, out_hbm.at[pl.ds(off, _C)])
                return carry

            lax.fori_loop(0, per_w // _C, body, 0)

        gather(iw_hbm, inp_hbm, o_inp, bpw)
        gather(ow_hbm, oemb_hbm, o_out, bpw)
        gather(nw_hbm, oemb_hbm, o_noise, npw)

    return k


_kernel = _build()


def kernel(input_words, output_words, noise_words, inp_embed, out_embed):
    iw = input_words.astype(jnp.int32)
    ow = output_words.astype(jnp.int32)
    nw = noise_words.reshape(-1).astype(jnp.int32)
    o_inp, o_out, o_noise = _kernel(iw, ow, nw, inp_embed, out_embed)
    return (o_inp, o_out, o_noise.reshape(B, S, D))


# speedup vs baseline: 1.3675x; 1.2082x over previous
import functools

import jax
import jax.numpy as jnp
from jax import lax
from jax.experimental import pallas as pl
from jax.experimental.pallas import tpu as pltpu
from jax.experimental.pallas import tpu_sc as plsc

B = 16384
D = 128
S = 5

_info = plsc.get_sparse_core_info()
_NC = _info.num_cores
_NS = _info.num_subcores
_NW = _NC * _NS

_C = 256


def _build():
    bpw = B // _NW
    npw = (B * S) // _NW
    tot = 2 * bpw + npw

    mesh = plsc.VectorSubcoreMesh(core_axis_name="c", subcore_axis_name="s")

    @functools.partial(
        pl.kernel,
        mesh=mesh,
        out_type=(
            jax.ShapeDtypeStruct((B, D), jnp.float32),
            jax.ShapeDtypeStruct((B, D), jnp.float32),
            jax.ShapeDtypeStruct((B * S, D), jnp.float32),
        ),
        scratch_types=[
            pltpu.VMEM((tot,), jnp.int32),
            pltpu.VMEM((_C, D), jnp.float32),
            pltpu.VMEM((_C, D), jnp.float32),
            pltpu.SemaphoreType.DMA,
            pltpu.SemaphoreType.DMA,
            pltpu.SemaphoreType.DMA,
            pltpu.SemaphoreType.DMA,
        ],
    )
    def k(iw_hbm, ow_hbm, nw_hbm, inp_hbm, oemb_hbm,
          o_inp, o_out, o_noise, idx_v, rows0, rows1,
          gsem0, gsem1, ssem0, ssem1):
        wid = lax.axis_index("s") * _NC + lax.axis_index("c")

        pltpu.sync_copy(iw_hbm.at[pl.ds(wid * bpw, bpw)],
                        idx_v.at[pl.ds(0, bpw)])
        pltpu.sync_copy(ow_hbm.at[pl.ds(wid * bpw, bpw)],
                        idx_v.at[pl.ds(bpw, bpw)])
        pltpu.sync_copy(nw_hbm.at[pl.ds(wid * npw, npw)],
                        idx_v.at[pl.ds(2 * bpw, npw)])

        chunks = []
        for i in range(bpw // _C):
            chunks.append((i * _C, inp_hbm, o_inp, wid * bpw + i * _C))
        for i in range(bpw // _C):
            chunks.append((bpw + i * _C, oemb_hbm, o_out, wid * bpw + i * _C))
        for i in range(npw // _C):
            chunks.append((2 * bpw + i * _C, oemb_hbm, o_noise,
                           wid * npw + i * _C))

        bufs = (rows0, rows1)
        gsems = (gsem0, gsem1)
        ssems = (ssem0, ssem1)
        n = len(chunks)
        g_h = [None] * n
        s_h = [None] * n

        for j in range(n):
            idx_off, table, out_hbm, out_off = chunks[j]
            if j >= 2:
                s_h[j - 2].wait()
            g_h[j] = pltpu.async_copy(
                table.at[idx_v.at[pl.ds(idx_off, _C)]],
                bufs[j % 2], gsems[j % 2])
            if j >= 1:
                p_idx, p_table, p_out, p_off = chunks[j - 1]
                g_h[j - 1].wait()
                s_h[j - 1] = pltpu.async_copy(
                    bufs[(j - 1) % 2],
                    p_out.at[pl.ds(p_off, _C)], ssems[(j - 1) % 2])

        g_h[n - 1].wait()
        _, _, l_out, l_off = chunks[n - 1]
        s_h[n - 1] = pltpu.async_copy(
            bufs[(n - 1) % 2], l_out.at[pl.ds(l_off, _C)], ssems[(n - 1) % 2])
        s_h[n - 2].wait()
        s_h[n - 1].wait()

    return k


_kernel = _build()


def kernel(input_words, output_words, noise_words, inp_embed, out_embed):
    iw = input_words.astype(jnp.int32)
    ow = output_words.astype(jnp.int32)
    nw = noise_words.reshape(-1).astype(jnp.int32)
    o_inp, o_out, o_noise = _kernel(iw, ow, nw, inp_embed, out_embed)
    return (o_inp, o_out, o_noise.reshape(B, S, D))


# speedup vs baseline: 2.1545x; 1.5755x over previous
import functools

import jax
import jax.numpy as jnp
from jax import lax
from jax.experimental import pallas as pl
from jax.experimental.pallas import tpu as pltpu
from jax.experimental.pallas import tpu_sc as plsc

B = 16384
D = 128
S = 5

_info = plsc.get_sparse_core_info()
_NC = _info.num_cores
_NS = _info.num_subcores
_NW = _NC * _NS

_C = 256


def _build():
    bpw = B // _NW
    tot = (2 + S) * bpw

    mesh = plsc.VectorSubcoreMesh(core_axis_name="c", subcore_axis_name="s")

    @functools.partial(
        pl.kernel,
        mesh=mesh,
        out_type=(
            jax.ShapeDtypeStruct((B, D), jnp.float32),
            jax.ShapeDtypeStruct((B, D), jnp.float32),
            jax.ShapeDtypeStruct((B, S, D), jnp.float32),
        ),
        scratch_types=[
            pltpu.VMEM((tot,), jnp.int32),
            pltpu.VMEM((_C, D), jnp.float32),
            pltpu.VMEM((_C, D), jnp.float32),
            pltpu.SemaphoreType.DMA,
            pltpu.SemaphoreType.DMA,
            pltpu.SemaphoreType.DMA,
            pltpu.SemaphoreType.DMA,
        ],
    )
    def k(iw_hbm, ow_hbm, nwt_hbm, inp_hbm, oemb_hbm,
          o_inp, o_out, o_noise, idx_v, rows0, rows1,
          gsem0, gsem1, ssem0, ssem1):
        wid = lax.axis_index("s") * _NC + lax.axis_index("c")
        base = wid * bpw

        pltpu.sync_copy(iw_hbm.at[pl.ds(base, bpw)],
                        idx_v.at[pl.ds(0, bpw)])
        pltpu.sync_copy(ow_hbm.at[pl.ds(base, bpw)],
                        idx_v.at[pl.ds(bpw, bpw)])
        for g in range(S):
            pltpu.sync_copy(nwt_hbm.at[pl.ds(g * B + base, bpw)],
                            idx_v.at[pl.ds((2 + g) * bpw, bpw)])

        chunks = []
        for i in range(bpw // _C):
            off = base + i * _C
            chunks.append((i * _C, inp_hbm,
                           lambda off=off: o_inp.at[pl.ds(off, _C)], False))
        for i in range(bpw // _C):
            off = base + i * _C
            chunks.append((bpw + i * _C, oemb_hbm,
                           lambda off=off: o_out.at[pl.ds(off, _C)], False))
        for g in range(S):
            for i in range(bpw // _C):
                off = base + i * _C
                chunks.append(((2 + g) * bpw + i * _C, oemb_hbm,
                               lambda off=off, g=g:
                               o_noise.at[pl.ds(off, _C), pl.ds(g, 1)],
                               True))

        bufs = (rows0, rows1)
        gsems = (gsem0, gsem1)
        ssems = (ssem0, ssem1)
        n = len(chunks)
        g_h = [None] * n
        s_h = [None] * n

        def store_src(j):
            buf = bufs[j % 2]
            return buf.reshape(_C, 1, D) if chunks[j][3] else buf

        for j in range(n):
            idx_off, table, _, _ = chunks[j]
            if j >= 2:
                s_h[j - 2].wait()
            g_h[j] = pltpu.async_copy(
                table.at[idx_v.at[pl.ds(idx_off, _C)]],
                bufs[j % 2], gsems[j % 2])
            if j >= 1:
                g_h[j - 1].wait()
                s_h[j - 1] = pltpu.async_copy(
                    store_src(j - 1), chunks[j - 1][2](),
                    ssems[(j - 1) % 2])

        g_h[n - 1].wait()
        s_h[n - 1] = pltpu.async_copy(
            store_src(n - 1), chunks[n - 1][2](), ssems[(n - 1) % 2])
        s_h[n - 2].wait()
        s_h[n - 1].wait()

    return k


_kernel = _build()


def kernel(input_words, output_words, noise_words, inp_embed, out_embed):
    iw = input_words.astype(jnp.int32)
    ow = output_words.astype(jnp.int32)
    nwt = noise_words.astype(jnp.int32).T.reshape(-1)
    return _kernel(iw, ow, nwt, inp_embed, out_embed)


# speedup vs baseline: 3.3148x; 1.5386x over previous
import functools

import jax
import jax.numpy as jnp
from jax import lax
from jax.experimental import pallas as pl
from jax.experimental.pallas import tpu as pltpu
from jax.experimental.pallas import tpu_sc as plsc

B = 16384
D = 128
S = 5

_info = plsc.get_sparse_core_info()
_NC = _info.num_cores
_NS = _info.num_subcores
_NW = _NC * _NS

_C = 256


def _build():
    bpw = B // _NW
    tot = (2 + S) * bpw

    mesh = plsc.VectorSubcoreMesh(core_axis_name="c", subcore_axis_name="s")

    @functools.partial(
        pl.kernel,
        mesh=mesh,
        out_type=(
            jax.ShapeDtypeStruct((B, D), jnp.float32),
            jax.ShapeDtypeStruct((B, D), jnp.float32),
            jax.ShapeDtypeStruct((S * B, D), jnp.float32),
        ),
        scratch_types=[
            pltpu.VMEM((tot,), jnp.int32),
            pltpu.VMEM((_C, D), jnp.float32),
            pltpu.VMEM((_C, D), jnp.float32),
            pltpu.SemaphoreType.DMA,
            pltpu.SemaphoreType.DMA,
            pltpu.SemaphoreType.DMA,
            pltpu.SemaphoreType.DMA,
        ],
    )
    def k(iw_hbm, ow_hbm, nwt_hbm, inp_hbm, oemb_hbm,
          o_inp, o_out, o_noise, idx_v, rows0, rows1,
          gsem0, gsem1, ssem0, ssem1):
        wid = lax.axis_index("s") * _NC + lax.axis_index("c")
        base = wid * bpw

        pltpu.sync_copy(iw_hbm.at[pl.ds(base, bpw)],
                        idx_v.at[pl.ds(0, bpw)])
        pltpu.sync_copy(ow_hbm.at[pl.ds(base, bpw)],
                        idx_v.at[pl.ds(bpw, bpw)])
        for g in range(S):
            pltpu.sync_copy(nwt_hbm.at[pl.ds(g * B + base, bpw)],
                            idx_v.at[pl.ds((2 + g) * bpw, bpw)])

        chunks = []
        for i in range(bpw // _C):
            chunks.append((i * _C, inp_hbm, o_inp, base + i * _C))
        for i in range(bpw // _C):
            chunks.append((bpw + i * _C, oemb_hbm, o_out, base + i * _C))
        for g in range(S):
            for i in range(bpw // _C):
                chunks.append(((2 + g) * bpw + i * _C, oemb_hbm, o_noise,
                               g * B + base + i * _C))

        bufs = (rows0, rows1)
        gsems = (gsem0, gsem1)
        ssems = (ssem0, ssem1)
        n = len(chunks)
        g_h = [None] * n
        s_h = [None] * n

        def store(j):
            _, _, out_hbm, row = chunks[j]
            return pltpu.async_copy(
                bufs[j % 2], out_hbm.at[pl.ds(row, _C)], ssems[j % 2])

        for j in range(n):
            idx_off, table, _, _ = chunks[j]
            if j >= 2:
                s_h[j - 2].wait()
            g_h[j] = pltpu.async_copy(
                table.at[idx_v.at[pl.ds(idx_off, _C)]],
                bufs[j % 2], gsems[j % 2])
            if j >= 1:
                g_h[j - 1].wait()
                s_h[j - 1] = store(j - 1)

        g_h[n - 1].wait()
        s_h[n - 1] = store(n - 1)
        s_h[n - 2].wait()
        s_h[n - 1].wait()

    return k


_kernel = _build()


def kernel(input_words, output_words, noise_words, inp_embed, out_embed):
    iw = input_words.astype(jnp.int32)
    ow = output_words.astype(jnp.int32)
    nwt = noise_words.astype(jnp.int32).T.reshape(-1)
    o_inp, o_out, o_noise = _kernel(iw, ow, nwt, inp_embed, out_embed)
    return (o_inp, o_out, o_noise.reshape(S, B, D).transpose(1, 0, 2))


# speedup vs baseline: 3.4194x; 1.0316x over previous
import functools

import jax
import jax.numpy as jnp
from jax import lax
from jax.experimental import pallas as pl
from jax.experimental.pallas import tpu as pltpu
from jax.experimental.pallas import tpu_sc as plsc

B = 16384
D = 128
S = 5

_info = plsc.get_sparse_core_info()
_NC = _info.num_cores
_NS = _info.num_subcores
_NW = _NC * _NS

_C = 256
_NB = 3


def _build():
    bpw = B // _NW
    tot = (2 + S) * bpw

    mesh = plsc.VectorSubcoreMesh(core_axis_name="c", subcore_axis_name="s")

    @functools.partial(
        pl.kernel,
        mesh=mesh,
        out_type=(
            jax.ShapeDtypeStruct((B, D), jnp.float32),
            jax.ShapeDtypeStruct((B, D), jnp.float32),
            jax.ShapeDtypeStruct((S * B, D), jnp.float32),
        ),
        scratch_types=[
            pltpu.VMEM((tot,), jnp.int32),
            pltpu.VMEM((_NB, _C, D), jnp.float32),
            pltpu.SemaphoreType.DMA,
        ] + [pltpu.SemaphoreType.DMA] * (2 * _NB),
    )
    def k(iw_hbm, ow_hbm, nwt_hbm, inp_hbm, oemb_hbm,
          o_inp, o_out, o_noise, idx_v, rows_v, isem, *sems):
        gsems = sems[:_NB]
        ssems = sems[_NB:]
        wid = lax.axis_index("s") * _NC + lax.axis_index("c")
        base = wid * bpw

        i_h = [
            pltpu.async_copy(iw_hbm.at[pl.ds(base, bpw)],
                             idx_v.at[pl.ds(0, bpw)], isem),
            pltpu.async_copy(ow_hbm.at[pl.ds(base, bpw)],
                             idx_v.at[pl.ds(bpw, bpw)], isem),
        ] + [
            pltpu.async_copy(nwt_hbm.at[pl.ds(g * B + base, bpw)],
                             idx_v.at[pl.ds((2 + g) * bpw, bpw)], isem)
            for g in range(S)
        ]
        for h in i_h:
            h.wait()

        chunks = []
        for i in range(bpw // _C):
            chunks.append((i * _C, inp_hbm, o_inp, base + i * _C))
        for i in range(bpw // _C):
            chunks.append((bpw + i * _C, oemb_hbm, o_out, base + i * _C))
        for g in range(S):
            for i in range(bpw // _C):
                chunks.append(((2 + g) * bpw + i * _C, oemb_hbm, o_noise,
                               g * B + base + i * _C))

        n = len(chunks)
        g_h = [None] * n
        s_h = [None] * n

        def store(j):
            _, _, out_hbm, row = chunks[j]
            return pltpu.async_copy(
                rows_v.at[j % _NB], out_hbm.at[pl.ds(row, _C)],
                ssems[j % _NB])

        for j in range(n):
            idx_off, table, _, _ = chunks[j]
            if j >= _NB:
                s_h[j - _NB].wait()
            g_h[j] = pltpu.async_copy(
                table.at[idx_v.at[pl.ds(idx_off, _C)]],
                rows_v.at[j % _NB], gsems[j % _NB])
            if j >= 1:
                g_h[j - 1].wait()
                s_h[j - 1] = store(j - 1)

        g_h[n - 1].wait()
        s_h[n - 1] = store(n - 1)
        for j in range(max(n - _NB + 1, 0), n):
            s_h[j].wait()

    return k


_kernel = _build()


def kernel(input_words, output_words, noise_words, inp_embed, out_embed):
    iw = input_words.astype(jnp.int32)
    ow = output_words.astype(jnp.int32)
    nwt = noise_words.astype(jnp.int32).T.reshape(-1)
    o_inp, o_out, o_noise = _kernel(iw, ow, nwt, inp_embed, out_embed)
    return (o_inp, o_out, o_noise.reshape(S, B, D).transpose(1, 0, 2))


# speedup vs baseline: 3.5337x; 1.0334x over previous
import functools

import jax
import jax.numpy as jnp
from jax import lax
from jax.experimental import pallas as pl
from jax.experimental.pallas import tpu as pltpu
from jax.experimental.pallas import tpu_sc as plsc

B = 16384
D = 128
S = 5

_info = plsc.get_sparse_core_info()
_NC = _info.num_cores
_NS = _info.num_subcores
_NW = _NC * _NS

_C = 256
_NB = 3
_K = 2


def _build():
    bpw = B // _NW
    tot = (2 + S) * bpw

    mesh = plsc.VectorSubcoreMesh(core_axis_name="c", subcore_axis_name="s")

    @functools.partial(
        pl.kernel,
        mesh=mesh,
        out_type=(
            jax.ShapeDtypeStruct((B, D), jnp.float32),
            jax.ShapeDtypeStruct((B, D), jnp.float32),
            jax.ShapeDtypeStruct((S * B, D), jnp.float32),
        ),
        scratch_types=[
            pltpu.VMEM((tot,), jnp.int32),
            pltpu.VMEM((_NB, _C, D), jnp.float32),
            pltpu.SemaphoreType.DMA,
        ] + [pltpu.SemaphoreType.DMA] * (2 * _NB),
    )
    def k(iw_hbm, ow_hbm, nwt_hbm, inp_hbm, oemb_hbm,
          o_inp, o_out, o_noise, idx_v, rows_v, isem, *sems):
        gsems = sems[:_NB]
        ssems = sems[_NB:]
        wid = lax.axis_index("s") * _NC + lax.axis_index("c")
        base = wid * bpw

        i_h = [
            pltpu.async_copy(iw_hbm.at[pl.ds(base, bpw)],
                             idx_v.at[pl.ds(0, bpw)], isem),
            pltpu.async_copy(ow_hbm.at[pl.ds(base, bpw)],
                             idx_v.at[pl.ds(bpw, bpw)], isem),
        ] + [
            pltpu.async_copy(nwt_hbm.at[pl.ds(g * B + base, bpw)],
                             idx_v.at[pl.ds((2 + g) * bpw, bpw)], isem)
            for g in range(S)
        ]

        chunks = []
        for i in range(bpw // _C):
            chunks.append((i * _C, inp_hbm, o_inp, base + i * _C))
        for i in range(bpw // _C):
            chunks.append((bpw + i * _C, oemb_hbm, o_out, base + i * _C))
        for g in range(S):
            for i in range(bpw // _C):
                chunks.append(((2 + g) * bpw + i * _C, oemb_hbm, o_noise,
                               g * B + base + i * _C))

        n = len(chunks)
        g_h = [None] * n
        s_h = [None] * n

        def store(j):
            _, _, out_hbm, row = chunks[j]
            return pltpu.async_copy(
                rows_v.at[j % _NB], out_hbm.at[pl.ds(row, _C)],
                ssems[j % _NB])

        staged = 0
        for j in range(n):
            idx_off, table, _, _ = chunks[j]
            region = idx_off // bpw
            while staged <= region:
                i_h[staged].wait()
                staged += 1
            if j >= _NB:
                s_h[j - _NB].wait()
            g_h[j] = pltpu.async_copy(
                table.at[idx_v.at[pl.ds(idx_off, _C)]],
                rows_v.at[j % _NB], gsems[j % _NB])
            if j >= _K:
                g_h[j - _K].wait()
                s_h[j - _K] = store(j - _K)

        for j in range(n - _K, n):
            g_h[j].wait()
            s_h[j] = store(j)
        for j in range(max(n - _NB, 0), n):
            if s_h[j] is not None:
                s_h[j].wait()

    return k


_kernel = _build()


def kernel(input_words, output_words, noise_words, inp_embed, out_embed):
    iw = input_words.astype(jnp.int32)
    ow = output_words.astype(jnp.int32)
    nwt = noise_words.astype(jnp.int32).T.reshape(-1)
    o_inp, o_out, o_noise = _kernel(iw, ow, nwt, inp_embed, out_embed)
    return (o_inp, o_out, o_noise.reshape(S, B, D).transpose(1, 0, 2))
